# baseline (speedup 1.0000x reference)
import functools

import jax
import jax.numpy as jnp
from jax import lax
from jax.experimental import pallas as pl
from jax.experimental.pallas import tpu as pltpu
from jax.experimental.pallas import tpu_sc as plsc

B = 16384
L = 20
EMBED = 32

NC, NS = 2, 16
NW = NC * NS
ROWS_W = B // NW
CHUNK = 32
NCHUNK = ROWS_W // CHUNK
TOK_W = ROWS_W * L
IDXC = 128
TOK_IDX_ROWS = TOK_W // IDXC
ART_IDX_ROWS = ROWS_W // IDXC
BURSTS = CHUNK * L // IDXC
TOKBUF_BYTES = CHUNK * L * EMBED * 4
OUTBUF_BYTES = CHUNK * 2 * EMBED * 4

_mesh = plsc.VectorSubcoreMesh(core_axis_name="c", subcore_axis_name="s")


@functools.partial(
    pl.kernel,
    out_type=jax.ShapeDtypeStruct((B, 2 * EMBED), jnp.float32),
    mesh=_mesh,
    compiler_params=pltpu.CompilerParams(
        needs_layout_passes=False, use_tc_tiling_on_sc=False),
    scratch_types=[
        pltpu.VMEM((ROWS_W, L), jnp.int32),
        pltpu.VMEM((TOK_IDX_ROWS, IDXC), jnp.int32),
        pltpu.VMEM((ART_IDX_ROWS, IDXC), jnp.int32),
        pltpu.VMEM((ROWS_W, EMBED), jnp.float32),
        pltpu.VMEM((2, CHUNK * L, EMBED), jnp.float32),
        pltpu.VMEM((ROWS_W,), jnp.float32),
        pltpu.VMEM((ROWS_W,), jnp.float32),
        pltpu.VMEM((1, EMBED), jnp.float32),
        pltpu.VMEM((2, CHUNK, 2 * EMBED), jnp.float32),
        pltpu.SemaphoreType.DMA,
        pltpu.SemaphoreType.DMA,
        pltpu.SemaphoreType.DMA,
        pltpu.SemaphoreType.DMA,
        pltpu.SemaphoreType.DMA,
    ],
)
def _article_kernel(tok_hbm, artid_hbm, art_tab, txt_tab, out_hbm,
                    tokstg, tokidx, artidx, artrows, tokrows, rcnt, zcnt,
                    row0, outbuf, sem_a, sem_b, sem_art, sem_oa, sem_ob):
    wid = lax.axis_index("s") * NC + lax.axis_index("c")
    base = wid * ROWS_W

    pltpu.sync_copy(tok_hbm.at[pl.ds(base, ROWS_W), :], tokstg)
    for j in range(ART_IDX_ROWS):
        pltpu.sync_copy(artid_hbm.at[pl.ds(base + j * IDXC, IDXC)],
                        artidx.at[j])
    pltpu.sync_copy(txt_tab.at[pl.ds(0, 1), :], row0)

    art_copies = []
    for j in range(ART_IDX_ROWS):
        art_copies.append(pltpu.async_copy(
            art_tab.at[artidx.at[j]],
            artrows.at[pl.ds(j * IDXC, IDXC), :],
            sem_art))

    lane = lax.iota(jnp.int32, 16)

    def cnt_body(g, carry):
        rows16 = g * 16 + lane
        p0 = rows16 * L
        cntf = jnp.zeros((16,), jnp.float32)
        for l in range(L):
            t = plsc.load_gather(tokstg, [rows16, jnp.zeros((16,), jnp.int32) + l])
            cntf = cntf + (t != 0).astype(jnp.float32)
            p = p0 + l
            plsc.store_scatter(
                tokidx,
                [jnp.right_shift(p, 7), jnp.bitwise_and(p, IDXC - 1)], t)
        rcnt[pl.ds(g * 16, 16)] = 1.0 / jnp.maximum(cntf, 1.0)
        zcnt[pl.ds(g * 16, 16)] = jnp.float32(L) - cntf
        return carry

    lax.fori_loop(0, ROWS_W // 16, cnt_body, 0)

    r0a = row0[0, pl.ds(0, 16)]
    r0b = row0[0, pl.ds(16, 16)]

    for cp in art_copies:
        cp.wait()

    def fire_chunk(c, par, sem):
        for j in range(BURSTS):
            pltpu.async_copy(
                txt_tab.at[tokidx.at[c * BURSTS + j]],
                tokrows.at[par, pl.ds(j * IDXC, IDXC), :],
                sem)

    def drain(ref_slice, sem):
        pltpu.make_async_copy(txt_tab.at[pl.ds(0, CHUNK * L), :],
                              ref_slice, sem).wait()

    def drain_out(par, sem):
        pltpu.make_async_copy(
            out_hbm.at[pl.ds(0, CHUNK), :], outbuf.at[par], sem).wait()

    def do_chunk(c, par):
        def row_body(r, rcarry):
            g = c * CHUNK + r
            acc0 = tokrows[par, r * L, pl.ds(0, 16)]
            acc1 = tokrows[par, r * L, pl.ds(16, 16)]
            for l in range(1, L):
                acc0 = acc0 + tokrows[par, r * L + l, pl.ds(0, 16)]
                acc1 = acc1 + tokrows[par, r * L + l, pl.ds(16, 16)]
            gidx = jnp.zeros((16,), jnp.int32) + g
            rc = plsc.load_gather(rcnt, [gidx])
            zc = plsc.load_gather(zcnt, [gidx])
            outbuf[par, r, pl.ds(0, 16)] = artrows[g, pl.ds(0, 16)]
            outbuf[par, r, pl.ds(16, 16)] = artrows[g, pl.ds(16, 16)]
            outbuf[par, r, pl.ds(32, 16)] = (acc0 - zc * r0a) * rc
            outbuf[par, r, pl.ds(48, 16)] = (acc1 - zc * r0b) * rc
            return rcarry

        lax.fori_loop(0, CHUNK, row_body, 0)

    fire_chunk(0, 0, sem_a)

    def pair_body(c2, carry):
        c = c2 * 2
        fire_chunk(c + 1, 1, sem_b)
        drain(tokrows.at[0], sem_a)

        @pl.when(c2 >= 1)
        def _():
            drain_out(0, sem_oa)

        do_chunk(c, 0)
        pltpu.async_copy(outbuf.at[0],
                         out_hbm.at[pl.ds(base + c * CHUNK, CHUNK), :],
                         sem_oa)

        @pl.when(c2 < NCHUNK // 2 - 1)
        def _():
            fire_chunk(c + 2, 0, sem_a)

        drain(tokrows.at[1], sem_b)

        @pl.when(c2 >= 1)
        def _():
            drain_out(1, sem_ob)

        do_chunk(c + 1, 1)
        pltpu.async_copy(outbuf.at[1],
                         out_hbm.at[pl.ds(base + (c + 1) * CHUNK, CHUNK), :],
                         sem_ob)
        return carry

    lax.fori_loop(0, NCHUNK // 2, pair_body, 0)
    drain_out(0, sem_oa)
    drain_out(1, sem_ob)


def kernel(article_id, prod_name_tokens, article_table, text_table):
    return _article_kernel(prod_name_tokens, article_id, article_table,
                           text_table)


# speedup vs baseline: 1.0911x; 1.0911x over previous
import functools

import jax
import jax.numpy as jnp
from jax import lax
from jax.experimental import pallas as pl
from jax.experimental.pallas import tpu as pltpu
from jax.experimental.pallas import tpu_sc as plsc

B = 16384
L = 20
EMBED = 32

NC, NS = 2, 16
NW = NC * NS
ROWS_W = B // NW
CHUNK = 32
NCHUNK = ROWS_W // CHUNK
TOK_W = ROWS_W * L
IDXC = 128
TOK_IDX_ROWS = TOK_W // IDXC
ART_IDX_ROWS = ROWS_W // IDXC
BURSTS = CHUNK * L // IDXC

_mesh = plsc.VectorSubcoreMesh(core_axis_name="c", subcore_axis_name="s")


@functools.partial(
    pl.kernel,
    out_type=jax.ShapeDtypeStruct((B, 2 * EMBED), jnp.float32),
    mesh=_mesh,
    compiler_params=pltpu.CompilerParams(
        needs_layout_passes=False, use_tc_tiling_on_sc=False),
    scratch_types=[
        pltpu.VMEM((TOK_IDX_ROWS, IDXC), jnp.int32),
        pltpu.VMEM((ART_IDX_ROWS, IDXC), jnp.int32),
        pltpu.VMEM((ROWS_W, EMBED), jnp.float32),
        pltpu.VMEM((2, CHUNK * L, EMBED), jnp.float32),
        pltpu.VMEM((ROWS_W,), jnp.float32),
        pltpu.VMEM((ROWS_W,), jnp.float32),
        pltpu.VMEM((1, EMBED), jnp.float32),
        pltpu.VMEM((2, CHUNK, EMBED), jnp.float32),
        pltpu.SemaphoreType.DMA,
        pltpu.SemaphoreType.DMA,
        pltpu.SemaphoreType.DMA,
        pltpu.SemaphoreType.DMA,
        pltpu.SemaphoreType.DMA,
    ],
)
def _article_kernel(tok_hbm, artid_hbm, art_tab, txt_tab, out_hbm,
                    tokidx, artidx, artrows, tokrows, rcnt, zcnt,
                    row0, outbuf, sem_a, sem_b, sem_art, sem_oa, sem_ob):
    wid = lax.axis_index("s") * NC + lax.axis_index("c")
    base = wid * ROWS_W

    pltpu.sync_copy(tok_hbm.at[pl.ds(wid * TOK_IDX_ROWS, TOK_IDX_ROWS), :],
                    tokidx)
    pltpu.sync_copy(artid_hbm.at[pl.ds(wid * ART_IDX_ROWS, ART_IDX_ROWS), :],
                    artidx)
    pltpu.sync_copy(txt_tab.at[pl.ds(0, 1), :], row0)

    art_copies = []
    for j in range(ART_IDX_ROWS):
        art_copies.append(pltpu.async_copy(
            art_tab.at[artidx.at[j]],
            artrows.at[pl.ds(j * IDXC, IDXC), :],
            sem_art))

    lane = lax.iota(jnp.int32, 16)

    def cnt_body(g, carry):
        p0 = (g * 16 + lane) * L
        cntf = jnp.zeros((16,), jnp.float32)
        for l in range(L):
            p = p0 + l
            t = plsc.load_gather(
                tokidx, [jnp.right_shift(p, 7), jnp.bitwise_and(p, IDXC - 1)])
            cntf = cntf + (t != 0).astype(jnp.float32)
        rcnt[pl.ds(g * 16, 16)] = 1.0 / jnp.maximum(cntf, 1.0)
        zcnt[pl.ds(g * 16, 16)] = jnp.float32(L) - cntf
        return carry

    lax.fori_loop(0, ROWS_W // 16, cnt_body, 0)

    r0a = row0[0, pl.ds(0, 16)]
    r0b = row0[0, pl.ds(16, 16)]

    for cp in art_copies:
        cp.wait()
    art_out = pltpu.async_copy(
        artrows, out_hbm.at[pl.ds(base, ROWS_W), pl.ds(0, EMBED)], sem_art)

    def fire_chunk(c, par, sem):
        for j in range(BURSTS):
            pltpu.async_copy(
                txt_tab.at[tokidx.at[c * BURSTS + j]],
                tokrows.at[par, pl.ds(j * IDXC, IDXC), :],
                sem)

    def drain(ref_slice, sem):
        pltpu.make_async_copy(txt_tab.at[pl.ds(0, CHUNK * L), :],
                              ref_slice, sem).wait()

    def drain_out(par, sem):
        pltpu.make_async_copy(
            txt_tab.at[pl.ds(0, CHUNK), :], outbuf.at[par], sem).wait()

    def do_chunk(c, par):
        def row_body(rr, rcarry):
            for half in range(2):
                r = rr * 2 + half
                g = c * CHUNK + r
                acc0 = tokrows[par, r * L, pl.ds(0, 16)]
                acc1 = tokrows[par, r * L, pl.ds(16, 16)]
                for l in range(1, L):
                    acc0 = acc0 + tokrows[par, r * L + l, pl.ds(0, 16)]
                    acc1 = acc1 + tokrows[par, r * L + l, pl.ds(16, 16)]
                gidx = jnp.zeros((16,), jnp.int32) + g
                rc = plsc.load_gather(rcnt, [gidx])
                zc = plsc.load_gather(zcnt, [gidx])
                outbuf[par, r, pl.ds(0, 16)] = (acc0 - zc * r0a) * rc
                outbuf[par, r, pl.ds(16, 16)] = (acc1 - zc * r0b) * rc
            return rcarry

        lax.fori_loop(0, CHUNK // 2, row_body, 0)

    fire_chunk(0, 0, sem_a)

    def pair_body(c2, carry):
        c = c2 * 2
        fire_chunk(c + 1, 1, sem_b)
        drain(tokrows.at[0], sem_a)

        @pl.when(c2 >= 1)
        def _():
            drain_out(0, sem_oa)

        do_chunk(c, 0)
        pltpu.async_copy(
            outbuf.at[0],
            out_hbm.at[pl.ds(base + c * CHUNK, CHUNK), pl.ds(EMBED, EMBED)],
            sem_oa)

        @pl.when(c2 < NCHUNK // 2 - 1)
        def _():
            fire_chunk(c + 2, 0, sem_a)

        drain(tokrows.at[1], sem_b)

        @pl.when(c2 >= 1)
        def _():
            drain_out(1, sem_ob)

        do_chunk(c + 1, 1)
        pltpu.async_copy(
            outbuf.at[1],
            out_hbm.at[pl.ds(base + (c + 1) * CHUNK, CHUNK),
                       pl.ds(EMBED, EMBED)],
            sem_ob)
        return carry

    lax.fori_loop(0, NCHUNK // 2, pair_body, 0)
    drain_out(0, sem_oa)
    drain_out(1, sem_ob)
    art_out.wait()


def kernel(article_id, prod_name_tokens, article_table, text_table):
    tok2d = prod_name_tokens.reshape(-1, IDXC)
    art2d = article_id.reshape(-1, IDXC)
    return _article_kernel(tok2d, art2d, article_table, text_table)


# speedup vs baseline: 1.1599x; 1.0630x over previous
import functools

import jax
import jax.numpy as jnp
from jax import lax
from jax.experimental import pallas as pl
from jax.experimental.pallas import tpu as pltpu
from jax.experimental.pallas import tpu_sc as plsc

B = 16384
L = 20
EMBED = 32

NC, NS = 2, 16
NW = NC * NS
ROWS_W = B // NW
CHUNK = 32
NCHUNK = ROWS_W // CHUNK
TOK_W = ROWS_W * L
IDXC = 128
TOK_IDX_ROWS = TOK_W // IDXC
ART_IDX_ROWS = ROWS_W // IDXC
BURSTS = CHUNK * L // IDXC

_mesh = plsc.VectorSubcoreMesh(core_axis_name="c", subcore_axis_name="s")


@functools.partial(
    pl.kernel,
    out_type=jax.ShapeDtypeStruct((B * L // IDXC, IDXC), jnp.int32),
    mesh=_mesh,
    compiler_params=pltpu.CompilerParams(
        needs_layout_passes=False, use_tc_tiling_on_sc=True),
    scratch_types=[
        pltpu.VMEM((ROWS_W, L), jnp.int32),
        pltpu.VMEM((TOK_IDX_ROWS, IDXC), jnp.int32),
    ],
)
def _repack_kernel(tok_hbm, out_hbm, tokstg, tokidx):
    wid = lax.axis_index("s") * NC + lax.axis_index("c")
    base = wid * ROWS_W
    pltpu.sync_copy(tok_hbm.at[pl.ds(base, ROWS_W), :], tokstg)
    lane = lax.iota(jnp.int32, 16)

    def body(g, carry):
        rows16 = g * 16 + lane
        p0 = rows16 * L
        for l in range(L):
            t = plsc.load_gather(
                tokstg, [rows16, jnp.zeros((16,), jnp.int32) + l])
            p = p0 + l
            plsc.store_scatter(
                tokidx,
                [jnp.right_shift(p, 7), jnp.bitwise_and(p, IDXC - 1)], t)
        return carry

    lax.fori_loop(0, ROWS_W // 16, body, 0)
    pltpu.sync_copy(tokidx,
                    out_hbm.at[pl.ds(wid * TOK_IDX_ROWS, TOK_IDX_ROWS), :])


@functools.partial(
    pl.kernel,
    out_type=jax.ShapeDtypeStruct((B, 2 * EMBED), jnp.float32),
    mesh=_mesh,
    compiler_params=pltpu.CompilerParams(
        needs_layout_passes=False, use_tc_tiling_on_sc=False),
    scratch_types=[
        pltpu.VMEM((TOK_IDX_ROWS, IDXC), jnp.int32),
        pltpu.VMEM((ART_IDX_ROWS, IDXC), jnp.int32),
        pltpu.VMEM((ROWS_W, EMBED), jnp.float32),
        pltpu.VMEM((2, CHUNK * L, EMBED), jnp.float32),
        pltpu.VMEM((ROWS_W,), jnp.float32),
        pltpu.VMEM((ROWS_W,), jnp.float32),
        pltpu.VMEM((1, EMBED), jnp.float32),
        pltpu.VMEM((2, CHUNK, EMBED), jnp.float32),
        pltpu.SemaphoreType.DMA,
        pltpu.SemaphoreType.DMA,
        pltpu.SemaphoreType.DMA,
        pltpu.SemaphoreType.DMA,
        pltpu.SemaphoreType.DMA,
    ],
)
def _article_kernel(tok_hbm, artid_hbm, art_tab, txt_tab, out_hbm,
                    tokidx, artidx, artrows, tokrows, rcnt, zcnt,
                    row0, outbuf, sem_a, sem_b, sem_art, sem_oa, sem_ob):
    wid = lax.axis_index("s") * NC + lax.axis_index("c")
    base = wid * ROWS_W

    pltpu.sync_copy(tok_hbm.at[pl.ds(wid * TOK_IDX_ROWS, TOK_IDX_ROWS), :],
                    tokidx)
    pltpu.sync_copy(artid_hbm.at[pl.ds(wid * ART_IDX_ROWS, ART_IDX_ROWS), :],
                    artidx)
    pltpu.sync_copy(txt_tab.at[pl.ds(0, 1), :], row0)

    art_copies = []
    for j in range(ART_IDX_ROWS):
        art_copies.append(pltpu.async_copy(
            art_tab.at[artidx.at[j]],
            artrows.at[pl.ds(j * IDXC, IDXC), :],
            sem_art))

    lane = lax.iota(jnp.int32, 16)

    def cnt_body(g, carry):
        p0 = (g * 16 + lane) * L
        cntf = jnp.zeros((16,), jnp.float32)
        for l in range(L):
            p = p0 + l
            t = plsc.load_gather(
                tokidx, [jnp.right_shift(p, 7), jnp.bitwise_and(p, IDXC - 1)])
            cntf = cntf + (t != 0).astype(jnp.float32)
        rcnt[pl.ds(g * 16, 16)] = 1.0 / jnp.maximum(cntf, 1.0)
        zcnt[pl.ds(g * 16, 16)] = jnp.float32(L) - cntf
        return carry

    lax.fori_loop(0, ROWS_W // 16, cnt_body, 0)

    r0a = row0[0, pl.ds(0, 16)]
    r0b = row0[0, pl.ds(16, 16)]

    for cp in art_copies:
        cp.wait()
    art_out = pltpu.async_copy(
        artrows, out_hbm.at[pl.ds(base, ROWS_W), pl.ds(0, EMBED)], sem_art)

    def fire_chunk(c, par, sem):
        for j in range(BURSTS):
            pltpu.async_copy(
                txt_tab.at[tokidx.at[c * BURSTS + j]],
                tokrows.at[par, pl.ds(j * IDXC, IDXC), :],
                sem)

    def drain(ref_slice, sem):
        pltpu.make_async_copy(txt_tab.at[pl.ds(0, CHUNK * L), :],
                              ref_slice, sem).wait()

    def drain_out(par, sem):
        pltpu.make_async_copy(
            txt_tab.at[pl.ds(0, CHUNK), :], outbuf.at[par], sem).wait()

    def do_chunk(c, par):
        def row_body(rr, rcarry):
            for half in range(2):
                r = rr * 2 + half
                g = c * CHUNK + r
                acc0 = tokrows[par, r * L, pl.ds(0, 16)]
                acc1 = tokrows[par, r * L, pl.ds(16, 16)]
                for l in range(1, L):
                    acc0 = acc0 + tokrows[par, r * L + l, pl.ds(0, 16)]
                    acc1 = acc1 + tokrows[par, r * L + l, pl.ds(16, 16)]
                gidx = jnp.zeros((16,), jnp.int32) + g
                rc = plsc.load_gather(rcnt, [gidx])
                zc = plsc.load_gather(zcnt, [gidx])
                outbuf[par, r, pl.ds(0, 16)] = (acc0 - zc * r0a) * rc
                outbuf[par, r, pl.ds(16, 16)] = (acc1 - zc * r0b) * rc
            return rcarry

        lax.fori_loop(0, CHUNK // 2, row_body, 0)

    fire_chunk(0, 0, sem_a)

    def pair_body(c2, carry):
        c = c2 * 2
        fire_chunk(c + 1, 1, sem_b)
        drain(tokrows.at[0], sem_a)

        @pl.when(c2 >= 1)
        def _():
            drain_out(0, sem_oa)

        do_chunk(c, 0)
        pltpu.async_copy(
            outbuf.at[0],
            out_hbm.at[pl.ds(base + c * CHUNK, CHUNK), pl.ds(EMBED, EMBED)],
            sem_oa)

        @pl.when(c2 < NCHUNK // 2 - 1)
        def _():
            fire_chunk(c + 2, 0, sem_a)

        drain(tokrows.at[1], sem_b)

        @pl.when(c2 >= 1)
        def _():
            drain_out(1, sem_ob)

        do_chunk(c + 1, 1)
        pltpu.async_copy(
            outbuf.at[1],
            out_hbm.at[pl.ds(base + (c + 1) * CHUNK, CHUNK),
                       pl.ds(EMBED, EMBED)],
            sem_ob)
        return carry

    lax.fori_loop(0, NCHUNK // 2, pair_body, 0)
    drain_out(0, sem_oa)
    drain_out(1, sem_ob)
    art_out.wait()


def kernel(article_id, prod_name_tokens, article_table, text_table):
    tok2d = _repack_kernel(prod_name_tokens)
    art2d = article_id.reshape(-1, IDXC)
    return _article_kernel(tok2d, art2d, article_table, text_table)
